# baseline (speedup 1.0000x reference)
import functools

import jax
import jax.numpy as jnp
from jax import lax
from jax.experimental import pallas as pl
from jax.experimental.pallas import tpu as pltpu
from jax.experimental.pallas import tpu_sc as plsc

VOCAB = 100000
HIDDEN = 128
B = 1024
L = 50
LPAD = 56

NC = 2
NS = 16
NW = NC * NS

TOTAL_P = B * LPAD
PER_W = TOTAL_P // NW
SEQ_PER_CHUNK = 8
CHUNK = SEQ_PER_CHUNK * LPAD
N_CHUNKS = PER_W // CHUNK

SEQ_PER_TC_BLOCK = 8


def _sc_gather(ids_pad, token_embedding):
    mesh = plsc.VectorSubcoreMesh(core_axis_name="c", subcore_axis_name="s")

    @functools.partial(
        pl.kernel,
        out_type=jax.ShapeDtypeStruct((B, LPAD, HIDDEN), jnp.float32),
        mesh=mesh,
        scratch_types=[
            pltpu.VMEM((PER_W,), jnp.int32),
            pltpu.VMEM((CHUNK, HIDDEN), jnp.float32),
            pltpu.VMEM((CHUNK, HIDDEN), jnp.float32),
            pltpu.SemaphoreType.DMA,
            pltpu.SemaphoreType.DMA,
            pltpu.SemaphoreType.DMA,
            pltpu.SemaphoreType.DMA,
        ],
    )
    def k(ids_hbm, table_hbm, out_hbm,
          idx_v, rows0, rows1, gsem0, gsem1, osem0, osem1):
        wid = lax.axis_index("s") * NC + lax.axis_index("c")
        base = wid * PER_W
        pltpu.sync_copy(ids_hbm.at[pl.ds(base, PER_W)], idx_v)

        rows = (rows0, rows1)
        gsems = (gsem0, gsem1)
        osems = (osem0, osem1)
        seq_base = wid * (PER_W // LPAD)

        def start_gather(g):
            return pltpu.async_copy(
                table_hbm.at[idx_v.at[pl.ds(g * CHUNK, CHUNK)]],
                rows[g % 2], gsems[g % 2])

        def start_out(g):
            rv = rows[g % 2]
            cps = []
            for s in range(SEQ_PER_CHUNK):
                cps.append(pltpu.async_copy(
                    rv.at[pl.ds(s * LPAD, LPAD)],
                    out_hbm.at[seq_base + g * SEQ_PER_CHUNK + s],
                    osems[g % 2]))
            return cps

        gcp = [None] * N_CHUNKS
        ocp = [None] * N_CHUNKS
        gcp[0] = start_gather(0)
        for g in range(N_CHUNKS):
            if g + 1 < N_CHUNKS:
                if g + 1 >= 2:
                    for cp in ocp[g - 1]:
                        cp.wait()
                gcp[g + 1] = start_gather(g + 1)
            gcp[g].wait()
            ocp[g] = start_out(g)
        for cp in ocp[N_CHUNKS - 2]:
            cp.wait()
        for cp in ocp[N_CHUNKS - 1]:
            cp.wait()

    return k(ids_pad, token_embedding)


def _tc_add_body(tok_ref, pos_ref, out_ref):
    out_ref[...] = tok_ref[:, pl.ds(0, L), :] + pos_ref[...][None, :, :]


def _tc_add(tok56, pos_block):
    grid = (B // SEQ_PER_TC_BLOCK,)
    return pl.pallas_call(
        _tc_add_body,
        grid=grid,
        in_specs=[
            pl.BlockSpec((SEQ_PER_TC_BLOCK, LPAD, HIDDEN), lambda i: (i, 0, 0)),
            pl.BlockSpec((L, HIDDEN), lambda i: (0, 0)),
        ],
        out_specs=pl.BlockSpec((SEQ_PER_TC_BLOCK, L, HIDDEN),
                               lambda i: (i, 0, 0)),
        out_shape=jax.ShapeDtypeStruct((B, L, HIDDEN), jnp.float32),
    )(tok56, pos_block)


def kernel(input_ids, token_embedding, position_embedding):
    ids_pad = jnp.pad(input_ids.astype(jnp.int32),
                      ((0, 0), (0, LPAD - L))).reshape(TOTAL_P)
    pos_block = position_embedding[0, :L, :]
    tok56 = _sc_gather(ids_pad, token_embedding)
    return _tc_add(tok56, pos_block)


# speedup vs baseline: 6.1999x; 6.1999x over previous
import functools

import jax
import jax.numpy as jnp
from jax import lax
from jax.experimental import pallas as pl
from jax.experimental.pallas import tpu as pltpu
from jax.experimental.pallas import tpu_sc as plsc

VOCAB = 100000
HIDDEN = 128
B = 1024
L = 50
NLANE = 16
NGRP = HIDDEN // NLANE

NC = 2
NS = 16
NW = NC * NS

TOTAL = B * L
PER_W = TOTAL // NW
SEQ_PER_W = PER_W // L
SEQ_PER_CHUNK = 8
CHUNK = SEQ_PER_CHUNK * L
N_CHUNKS = PER_W // CHUNK


def _sc_embed(ids_flat, token_embedding, pos_block):
    mesh = plsc.VectorSubcoreMesh(core_axis_name="c", subcore_axis_name="s")

    @functools.partial(
        pl.kernel,
        out_type=jax.ShapeDtypeStruct((B, L, HIDDEN), jnp.float32),
        mesh=mesh,
        scratch_types=[
            pltpu.VMEM((PER_W,), jnp.int32),
            pltpu.VMEM((L, HIDDEN), jnp.float32),
            pltpu.VMEM((CHUNK, HIDDEN), jnp.float32),
            pltpu.VMEM((CHUNK, HIDDEN), jnp.float32),
            pltpu.SemaphoreType.DMA,
            pltpu.SemaphoreType.DMA,
            pltpu.SemaphoreType.DMA,
            pltpu.SemaphoreType.DMA,
            pltpu.SemaphoreType.DMA,
        ],
    )
    def k(ids_hbm, table_hbm, pos_hbm, out_hbm,
          idx_v, pos_v, rows0, rows1, gsem0, gsem1, osem0, osem1, psem):
        wid = lax.axis_index("s") * NC + lax.axis_index("c")
        base = wid * PER_W
        pcp = pltpu.async_copy(pos_hbm, pos_v, psem)
        pltpu.sync_copy(ids_hbm.at[pl.ds(base, PER_W)], idx_v)

        rows = (rows0, rows1)
        gsems = (gsem0, gsem1)
        osems = (osem0, osem1)
        seq_base = wid * SEQ_PER_W

        def start_gather(g):
            return pltpu.async_copy(
                table_hbm.at[idx_v.at[pl.ds(g * CHUNK, CHUNK)]],
                rows[g % 2], gsems[g % 2])

        def add_pos(g):
            rv = rows[g % 2]

            def body(p, carry):
                regs = [pos_v[p, pl.ds(c * NLANE, NLANE)] for c in range(NGRP)]
                for s in range(SEQ_PER_CHUNK):
                    for c in range(NGRP):
                        plsc.addupdate(
                            rv.at[s * L + p, pl.ds(c * NLANE, NLANE)], regs[c])
                return carry

            lax.fori_loop(0, L, body, 0, unroll=False)

        def start_out(g):
            rv = rows[g % 2]
            return [pltpu.async_copy(
                        rv.at[pl.ds(s * L, L)],
                        out_hbm.at[seq_base + g * SEQ_PER_CHUNK + s],
                        osems[g % 2])
                    for s in range(SEQ_PER_CHUNK)]

        gcp = [None] * N_CHUNKS
        ocp = [None] * N_CHUNKS
        gcp[0] = start_gather(0)
        pcp.wait()
        for g in range(N_CHUNKS):
            if g + 1 < N_CHUNKS:
                if g >= 1:
                    for cp in ocp[g - 1]:
                        cp.wait()
                gcp[g + 1] = start_gather(g + 1)
            gcp[g].wait()
            add_pos(g)
            ocp[g] = start_out(g)
        for cp in ocp[N_CHUNKS - 2]:
            cp.wait()
        for cp in ocp[N_CHUNKS - 1]:
            cp.wait()

    return k(ids_flat, token_embedding, pos_block)


def kernel(input_ids, token_embedding, position_embedding):
    ids_flat = input_ids.astype(jnp.int32).reshape(TOTAL)
    pos_block = position_embedding[0, :L, :]
    return _sc_embed(ids_flat, token_embedding, pos_block)


# speedup vs baseline: 8.5697x; 1.3822x over previous
import functools

import jax
import jax.numpy as jnp
from jax import lax
from jax.experimental import pallas as pl
from jax.experimental.pallas import tpu as pltpu
from jax.experimental.pallas import tpu_sc as plsc

VOCAB = 100000
HIDDEN = 128
B = 1024
L = 50
NLANE = 16
NGRP = HIDDEN // NLANE

NC = 2
NS = 16
NW = NC * NS

TOTAL = B * L
PER_W = TOTAL // NW
SEQS = B // NW
P_CHUNK = 10
N_CHUNKS = L // P_CHUNK
CHUNK = P_CHUNK * SEQS


def _sc_embed(ids_perm, token_embedding, pos_block):
    mesh = plsc.VectorSubcoreMesh(core_axis_name="c", subcore_axis_name="s")

    @functools.partial(
        pl.kernel,
        out_type=jax.ShapeDtypeStruct((L, B, HIDDEN), jnp.float32),
        mesh=mesh,
        scratch_types=[
            pltpu.VMEM((PER_W,), jnp.int32),
            pltpu.VMEM((L, HIDDEN), jnp.float32),
            pltpu.VMEM((CHUNK, HIDDEN), jnp.float32),
            pltpu.VMEM((CHUNK, HIDDEN), jnp.float32),
            pltpu.SemaphoreType.DMA,
            pltpu.SemaphoreType.DMA,
            pltpu.SemaphoreType.DMA,
            pltpu.SemaphoreType.DMA,
            pltpu.SemaphoreType.DMA,
        ],
    )
    def k(ids_hbm, table_hbm, pos_hbm, out_hbm,
          idx_v, pos_v, rows0, rows1, gsem0, gsem1, osem0, osem1, psem):
        wid = lax.axis_index("s") * NC + lax.axis_index("c")
        base = wid * PER_W
        pcp = pltpu.async_copy(pos_hbm, pos_v, psem)
        pltpu.sync_copy(ids_hbm.at[pl.ds(base, PER_W)], idx_v)

        rows = (rows0, rows1)
        gsems = (gsem0, gsem1)
        osems = (osem0, osem1)
        seq_base = wid * SEQS

        def start_gather(g):
            return pltpu.async_copy(
                table_hbm.at[idx_v.at[pl.ds(g * CHUNK, CHUNK)]],
                rows[g % 2], gsems[g % 2])

        def add_pos(g):
            rv = rows[g % 2]

            def body(i, carry):
                p = g * P_CHUNK + i
                regs = [pos_v[p, pl.ds(c * NLANE, NLANE)] for c in range(NGRP)]
                row0 = i * SEQS
                for s in range(SEQS):
                    for c in range(NGRP):
                        plsc.addupdate(
                            rv.at[row0 + s, pl.ds(c * NLANE, NLANE)], regs[c])
                return carry

            lax.fori_loop(0, P_CHUNK, body, 0, unroll=False)

        def start_out(g):
            rv = rows[g % 2]
            return [pltpu.async_copy(
                        rv.at[pl.ds(i * SEQS, SEQS)],
                        out_hbm.at[g * P_CHUNK + i, pl.ds(seq_base, SEQS)],
                        osems[g % 2])
                    for i in range(P_CHUNK)]

        gcp = [None] * N_CHUNKS
        ocp = [None] * N_CHUNKS
        gcp[0] = start_gather(0)
        pcp.wait()
        for g in range(N_CHUNKS):
            if g + 1 < N_CHUNKS:
                if g >= 1:
                    for cp in ocp[g - 1]:
                        cp.wait()
                gcp[g + 1] = start_gather(g + 1)
            gcp[g].wait()
            add_pos(g)
            ocp[g] = start_out(g)
        for cp in ocp[N_CHUNKS - 2]:
            cp.wait()
        for cp in ocp[N_CHUNKS - 1]:
            cp.wait()

    return k(ids_perm, token_embedding, pos_block)


def kernel(input_ids, token_embedding, position_embedding):
    ids_perm = (input_ids.astype(jnp.int32)
                .reshape(NW, SEQS, N_CHUNKS, P_CHUNK)
                .transpose(0, 2, 3, 1)
                .reshape(TOTAL))
    pos_block = position_embedding[0, :L, :]
    out_t = _sc_embed(ids_perm, token_embedding, pos_block)
    return jnp.transpose(out_t, (1, 0, 2))
